# baseline (speedup 1.0000x reference)
import functools

import jax
import jax.numpy as jnp
from jax import lax
from jax.experimental import pallas as pl
from jax.experimental.pallas import tpu as pltpu
from jax.experimental.pallas import tpu_sc as plsc
from jax.scipy.linalg import block_diag

N_NODES = 10000
N_EDGES = 160000
IN_DIM = 32
OUT_DIM = 32
EDGE_DIM = 16
INNER_DIM = 64

NC = 2
NS = 16
NW = NC * NS
CH = 128
NCH = 40
EPW = NCH * CH
EP = NW * EPW
NP = 10240
RPT = NP // NS

BE = 2048
BE4 = BE // 4
HALF = EPW // 2
KF = HALF // CH

_mesh = plsc.VectorSubcoreMesh(core_axis_name="c", subcore_axis_name="s")


@functools.partial(
    pl.kernel,
    out_type=jax.ShapeDtypeStruct((EP, IN_DIM), jnp.float32),
    mesh=_mesh,
    scratch_types=[
        pltpu.VMEM((NCH, CH), jnp.int32),
        pltpu.VMEM((HALF, IN_DIM), jnp.float32),
        pltpu.SemaphoreType.DMA,
    ],
    compiler_params=pltpu.CompilerParams(use_tc_tiling_on_sc=False),
)
def _sc_gather(x_hbm, src_hbm, xj_hbm, idx_v, rows_v, sem):
    wid = lax.axis_index("s") * NC + lax.axis_index("c")
    base = wid * EPW
    pltpu.sync_copy(src_hbm.at[wid], idx_v)

    def half_step(h, carry):
        descs = []
        for t in range(KF):
            descs.append(pltpu.async_copy(
                x_hbm.at[idx_v.at[h * KF + t]],
                rows_v.at[pl.ds(t * CH, CH)], sem))
        for d in descs:
            d.wait()
        pltpu.sync_copy(rows_v, xj_hbm.at[pl.ds(base + h * HALF, HALF)])
        return carry

    lax.fori_loop(0, EPW // HALF, half_step, 0)


@functools.partial(
    pl.kernel,
    out_type=jax.ShapeDtypeStruct((NC, NP, OUT_DIM), jnp.float32),
    mesh=_mesh,
    scratch_types=[
        pltpu.VMEM((NCH, CH), jnp.int32),
        pltpu.VMEM((HALF, OUT_DIM), jnp.float32),
        pltpu.VMEM_SHARED((NP, OUT_DIM), jnp.float32),
        pltpu.SemaphoreType.DMA,
    ],
    compiler_params=pltpu.CompilerParams(use_tc_tiling_on_sc=False),
)
def _sc_scatter(m_hbm, dst_hbm, zero_hbm, agg_hbm, idx_v, rows_v, shared, sem):
    cid = lax.axis_index("c")
    sid = lax.axis_index("s")
    wid = sid * NC + cid
    base = wid * EPW
    r0 = sid * RPT
    pltpu.sync_copy(zero_hbm.at[pl.ds(r0, RPT)], shared.at[pl.ds(r0, RPT)])
    pltpu.sync_copy(dst_hbm.at[wid], idx_v)
    plsc.subcore_barrier()

    def half_step(h, carry):
        pltpu.sync_copy(m_hbm.at[pl.ds(base + h * HALF, HALF)], rows_v)
        descs = []
        for t in range(KF):
            descs.append(pltpu.async_copy(
                rows_v.at[pl.ds(t * CH, CH)],
                shared.at[idx_v.at[h * KF + t]], sem, add=True))
        for d in descs:
            d.wait()
        return carry

    lax.fori_loop(0, EPW // HALF, half_step, 0)
    plsc.subcore_barrier()
    pltpu.sync_copy(shared.at[pl.ds(r0, RPT)], agg_hbm.at[cid, pl.ds(r0, RPT)])


def _tc_edge_body(edge_ref, xj_any, w1_ref, b1_ref, w2_ref, b2_ref, r_ref,
                  m_ref, xj_buf, sems):
    i = pl.program_id(0)
    n = pl.num_programs(0)

    def xj_copy(blk, slot):
        return pltpu.make_async_copy(
            xj_any.at[pl.ds(blk * BE4, BE4)], xj_buf.at[slot], sems.at[slot])

    @pl.when(i == 0)
    def _():
        xj_copy(0, 0).start()

    @pl.when(i + 1 < n)
    def _():
        xj_copy(i + 1, lax.rem(i + 1, 2)).start()

    slot = lax.rem(i, 2)
    xj_copy(i, slot).wait()
    xj_blk = xj_buf[slot]

    h = jnp.dot(edge_ref[...], w1_ref[...],
                preferred_element_type=jnp.float32) + b1_ref[...]
    xjb = xj_blk.astype(jnp.bfloat16)
    h2 = jnp.dot(h.astype(jnp.bfloat16), w2_ref[...],
                 preferred_element_type=jnp.float32)
    xju = jnp.dot(xjb, r_ref[...], preferred_element_type=jnp.float32)
    p = xju * h2
    w = IN_DIM * OUT_DIM
    while w > OUT_DIM:
        half = w // 2
        p = jnp.concatenate(
            [p[:, q * w:q * w + half] + p[:, q * w + half:(q + 1) * w]
             for q in range(4)], axis=1)
        w = half
    m_ref[...] = p + jnp.dot(xjb, b2_ref[...],
                             preferred_element_type=jnp.float32)


def _tc_final_body(a0_ref, a1_ref, x_ref, root_ref, bias_ref, o_ref):
    xr = jnp.dot(x_ref[...], root_ref[...], preferred_element_type=jnp.float32)
    acc = a0_ref[...] + a1_ref[...] + xr + bias_ref[...]
    o_ref[...] = jnp.maximum(acc, 0.0)


def kernel(x, adj, edge, W1, b1, W2, b2, root, bias):
    src = adj[0].astype(jnp.int32)
    dst = adj[1].astype(jnp.int32)
    pad = EP - N_EDGES
    src_p = jnp.concatenate(
        [src, jnp.arange(pad, dtype=jnp.int32) % N_NODES]).reshape(NW, NCH, CH)
    dst_p = jnp.concatenate(
        [dst, N_NODES + jnp.arange(pad, dtype=jnp.int32) % (NP - N_NODES)]
    ).reshape(NW, NCH, CH)
    edge4 = jnp.concatenate(
        [edge, jnp.zeros((pad, EDGE_DIM), jnp.float32)],
        axis=0).reshape(EP // 4, 4 * EDGE_DIM)

    xj4 = _sc_gather(x, src_p).reshape(EP // 4, 4 * IN_DIM)

    r_mat = jnp.repeat(jnp.eye(IN_DIM, dtype=jnp.bfloat16), OUT_DIM, axis=1)
    w1_bd = block_diag(W1, W1, W1, W1)
    b1_t = jnp.tile(b1, 4).reshape(1, 4 * INNER_DIM)
    w2_bd = block_diag(W2, W2, W2, W2).astype(jnp.bfloat16)
    r_bd = block_diag(r_mat, r_mat, r_mat, r_mat)
    b2m = b2.reshape(IN_DIM, OUT_DIM).astype(jnp.bfloat16)
    b2_bd = block_diag(b2m, b2m, b2m, b2m)

    m4 = pl.pallas_call(
        _tc_edge_body,
        grid=(EP // BE,),
        in_specs=[
            pl.BlockSpec((BE4, 4 * EDGE_DIM), lambda i: (i, 0)),
            pl.BlockSpec(memory_space=pl.ANY),
            pl.BlockSpec((4 * EDGE_DIM, 4 * INNER_DIM), lambda i: (0, 0)),
            pl.BlockSpec((1, 4 * INNER_DIM), lambda i: (0, 0)),
            pl.BlockSpec((4 * INNER_DIM, 4 * IN_DIM * OUT_DIM),
                         lambda i: (0, 0)),
            pl.BlockSpec((4 * IN_DIM, 4 * OUT_DIM), lambda i: (0, 0)),
            pl.BlockSpec((4 * IN_DIM, 4 * IN_DIM * OUT_DIM), lambda i: (0, 0)),
        ],
        out_specs=pl.BlockSpec((BE4, 4 * OUT_DIM), lambda i: (i, 0)),
        out_shape=jax.ShapeDtypeStruct((EP // 4, 4 * OUT_DIM), jnp.float32),
        scratch_shapes=[
            pltpu.VMEM((2, BE4, 4 * IN_DIM), jnp.float32),
            pltpu.SemaphoreType.DMA((2,)),
        ],
    )(edge4, xj4, w1_bd, b1_t, w2_bd, b2_bd, r_bd)

    agg = _sc_scatter(m4.reshape(EP, OUT_DIM), dst_p,
                      jnp.zeros((NP, OUT_DIM), jnp.float32))

    out = pl.pallas_call(
        _tc_final_body,
        out_shape=jax.ShapeDtypeStruct((N_NODES, OUT_DIM), jnp.float32),
    )(agg[0, :N_NODES], agg[1, :N_NODES], x, root,
      bias.reshape(1, OUT_DIM))
    return out


# speedup vs baseline: 1.1606x; 1.1606x over previous
import functools

import jax
import jax.numpy as jnp
from jax import lax
from jax.experimental import pallas as pl
from jax.experimental.pallas import tpu as pltpu
from jax.experimental.pallas import tpu_sc as plsc
from jax.scipy.linalg import block_diag

N_NODES = 10000
N_EDGES = 160000
IN_DIM = 32
OUT_DIM = 32
EDGE_DIM = 16
INNER_DIM = 64

NC = 2
NS = 16
NW = NC * NS
CH = 125
NCH = 40
EPW = NCH * CH
NP = 10240
RPT = NP // NS

BE = 1600
BE4 = BE // 4
HALF = EPW // 2
KF = HALF // CH

_mesh = plsc.VectorSubcoreMesh(core_axis_name="c", subcore_axis_name="s")


@functools.partial(
    pl.kernel,
    out_type=jax.ShapeDtypeStruct((N_EDGES, IN_DIM), jnp.float32),
    mesh=_mesh,
    scratch_types=[
        pltpu.VMEM((NCH, CH), jnp.int32),
        pltpu.VMEM((HALF, IN_DIM), jnp.float32),
        pltpu.SemaphoreType.DMA,
    ],
    compiler_params=pltpu.CompilerParams(use_tc_tiling_on_sc=False),
)
def _sc_gather(x_hbm, src_hbm, xj_hbm, idx_v, rows_v, sem):
    wid = lax.axis_index("s") * NC + lax.axis_index("c")
    base = wid * EPW
    pltpu.sync_copy(src_hbm.at[wid], idx_v)

    def half_step(h, carry):
        descs = []
        for t in range(KF):
            descs.append(pltpu.async_copy(
                x_hbm.at[idx_v.at[h * KF + t]],
                rows_v.at[pl.ds(t * CH, CH)], sem))
        for d in descs:
            d.wait()
        pltpu.sync_copy(rows_v, xj_hbm.at[pl.ds(base + h * HALF, HALF)])
        return carry

    lax.fori_loop(0, EPW // HALF, half_step, 0)


@functools.partial(
    pl.kernel,
    out_type=jax.ShapeDtypeStruct((NC, NP, OUT_DIM), jnp.float32),
    mesh=_mesh,
    scratch_types=[
        pltpu.VMEM((NCH, CH), jnp.int32),
        pltpu.VMEM((HALF, OUT_DIM), jnp.float32),
        pltpu.VMEM_SHARED((NP, OUT_DIM), jnp.float32),
        pltpu.SemaphoreType.DMA,
    ],
    compiler_params=pltpu.CompilerParams(use_tc_tiling_on_sc=False),
)
def _sc_scatter(m_hbm, dst_hbm, zero_hbm, agg_hbm, idx_v, rows_v, shared, sem):
    cid = lax.axis_index("c")
    sid = lax.axis_index("s")
    wid = sid * NC + cid
    base = wid * EPW
    r0 = sid * RPT
    pltpu.sync_copy(zero_hbm.at[pl.ds(r0, RPT)], shared.at[pl.ds(r0, RPT)])
    pltpu.sync_copy(dst_hbm.at[wid], idx_v)
    plsc.subcore_barrier()

    def half_step(h, carry):
        pltpu.sync_copy(m_hbm.at[pl.ds(base + h * HALF, HALF)], rows_v)
        descs = []
        for t in range(KF):
            descs.append(pltpu.async_copy(
                rows_v.at[pl.ds(t * CH, CH)],
                shared.at[idx_v.at[h * KF + t]], sem, add=True))
        for d in descs:
            d.wait()
        return carry

    lax.fori_loop(0, EPW // HALF, half_step, 0)
    plsc.subcore_barrier()
    pltpu.sync_copy(shared.at[pl.ds(r0, RPT)], agg_hbm.at[cid, pl.ds(r0, RPT)])


def _tc_edge_body(edge_ref, xj_ref, w12_ref, r_ref, bc_ref, m_ref):
    xjb = xj_ref[...].astype(jnp.bfloat16)
    h2 = jnp.dot(edge_ref[...].astype(jnp.bfloat16), w12_ref[...],
                 preferred_element_type=jnp.float32)
    xju = jnp.dot(xjb, r_ref[...], preferred_element_type=jnp.float32)
    p = xju * h2
    w = IN_DIM * OUT_DIM
    while w > OUT_DIM:
        half = w // 2
        p = jnp.concatenate(
            [p[:, q * w:q * w + half] + p[:, q * w + half:(q + 1) * w]
             for q in range(4)], axis=1)
        w = half
    m_ref[...] = p + jnp.dot(xjb, bc_ref[...],
                             preferred_element_type=jnp.float32)


def _tc_final_body(a0_ref, a1_ref, x_ref, root_ref, bias_ref, o_ref):
    xr = jnp.dot(x_ref[...], root_ref[...], preferred_element_type=jnp.float32)
    acc = a0_ref[...] + a1_ref[...] + xr + bias_ref[...]
    o_ref[...] = jnp.maximum(acc, 0.0)


def kernel(x, adj, edge, W1, b1, W2, b2, root, bias):
    src_p = adj[0].astype(jnp.int32).reshape(NW, NCH, CH)
    dst_p = adj[1].astype(jnp.int32).reshape(NW, NCH, CH)

    xj4 = _sc_gather(x, src_p).reshape(N_EDGES // 4, 4 * IN_DIM)

    w12 = jnp.dot(W1, W2).astype(jnp.bfloat16)
    w12_bd = block_diag(w12, w12, w12, w12)
    r_mat = jnp.repeat(jnp.eye(IN_DIM, dtype=jnp.bfloat16), OUT_DIM, axis=1)
    r_bd = block_diag(r_mat, r_mat, r_mat, r_mat)
    bc = (jnp.dot(b1, W2) + b2).reshape(IN_DIM, OUT_DIM).astype(jnp.bfloat16)
    bc_bd = block_diag(bc, bc, bc, bc)
    edge4 = edge.reshape(N_EDGES // 4, 4 * EDGE_DIM)

    m4 = pl.pallas_call(
        _tc_edge_body,
        grid=(N_EDGES // BE,),
        in_specs=[
            pl.BlockSpec((BE4, 4 * EDGE_DIM), lambda i: (i, 0)),
            pl.BlockSpec((BE4, 4 * IN_DIM), lambda i: (i, 0)),
            pl.BlockSpec((4 * EDGE_DIM, 4 * IN_DIM * OUT_DIM),
                         lambda i: (0, 0)),
            pl.BlockSpec((4 * IN_DIM, 4 * IN_DIM * OUT_DIM), lambda i: (0, 0)),
            pl.BlockSpec((4 * IN_DIM, 4 * OUT_DIM), lambda i: (0, 0)),
        ],
        out_specs=pl.BlockSpec((BE4, 4 * OUT_DIM), lambda i: (i, 0)),
        out_shape=jax.ShapeDtypeStruct((N_EDGES // 4, 4 * OUT_DIM),
                                       jnp.float32),
    )(edge4, xj4, w12_bd, r_bd, bc_bd)

    agg = _sc_scatter(m4.reshape(N_EDGES, OUT_DIM), dst_p,
                      jnp.zeros((NP, OUT_DIM), jnp.float32))

    out = pl.pallas_call(
        _tc_final_body,
        out_shape=jax.ShapeDtypeStruct((N_NODES, OUT_DIM), jnp.float32),
    )(agg[0, :N_NODES], agg[1, :N_NODES], x, root,
      bias.reshape(1, OUT_DIM))
    return out
